# Initial kernel scaffold; baseline (speedup 1.0000x reference)
import jax
import jax.numpy as jnp
from jax.experimental import pallas as pl


def kernel(scores):
    raise NotImplementedError("write your pallas kernel here")


# speedup vs baseline: 11.6828x; 11.6828x over previous
import jax
import jax.numpy as jnp
from jax.experimental import pallas as pl

FILTER_THRESHOLD = 0.95
FILTER_FACTOR = 0.5

_ROWS_PER_BLOCK = 256


def _filter_block(scores_ref, out_ref):
    i = pl.program_id(0)
    blk = scores_ref[...]
    rows, cols = blk.shape
    row_iota = jax.lax.broadcasted_iota(jnp.int32, (rows, cols), 0)
    col_iota = jax.lax.broadcasted_iota(jnp.int32, (rows, cols), 1)
    is_diag = col_iota == row_iota + i * rows
    diag = jnp.max(jnp.where(is_diag, blk, -jnp.inf), axis=1, keepdims=True)
    thresh = FILTER_THRESHOLD * diag
    mask = (blk > thresh) & jnp.logical_not(is_diag)
    out_ref[...] = jnp.where(mask, blk * FILTER_FACTOR, blk)


def kernel(scores):
    B = scores.shape[0]
    rows = _ROWS_PER_BLOCK
    grid = (B // rows,)
    return pl.pallas_call(
        _filter_block,
        grid=grid,
        in_specs=[pl.BlockSpec((rows, B), lambda i: (i, 0))],
        out_specs=pl.BlockSpec((rows, B), lambda i: (i, 0)),
        out_shape=jax.ShapeDtypeStruct(scores.shape, scores.dtype),
    )(scores)


# speedup vs baseline: 13.8592x; 1.1863x over previous
import jax
import jax.numpy as jnp
from jax.experimental import pallas as pl

FILTER_THRESHOLD = 0.95
FILTER_FACTOR = 0.5

_ROWS_PER_BLOCK = 256


def _filter_block(scores_ref, out_ref):
    i = pl.program_id(0)
    blk = scores_ref[...]
    rows = blk.shape[0]
    sub = scores_ref[:, pl.ds(i * rows, rows)]
    r_iota = jax.lax.broadcasted_iota(jnp.int32, (rows, rows), 0)
    c_iota = jax.lax.broadcasted_iota(jnp.int32, (rows, rows), 1)
    eq = r_iota == c_iota
    diag = jnp.max(jnp.where(eq, sub, -jnp.inf), axis=1, keepdims=True)
    thresh = FILTER_THRESHOLD * diag
    out_ref[...] = jnp.where(blk > thresh, blk * FILTER_FACTOR, blk)
    sub_filtered = jnp.where(sub > thresh, sub * FILTER_FACTOR, sub)
    out_ref[:, pl.ds(i * rows, rows)] = jnp.where(eq, sub, sub_filtered)


def kernel(scores):
    B = scores.shape[0]
    rows = _ROWS_PER_BLOCK
    grid = (B // rows,)
    return pl.pallas_call(
        _filter_block,
        grid=grid,
        in_specs=[pl.BlockSpec((rows, B), lambda i: (i, 0))],
        out_specs=pl.BlockSpec((rows, B), lambda i: (i, 0)),
        out_shape=jax.ShapeDtypeStruct(scores.shape, scores.dtype),
    )(scores)


# speedup vs baseline: 14.1546x; 1.0213x over previous
import jax
import jax.numpy as jnp
from jax.experimental import pallas as pl

FILTER_THRESHOLD = 0.95
FILTER_FACTOR = 0.5

_ROWS_PER_BLOCK = 512


def _filter_block(scores_ref, out_ref):
    i = pl.program_id(0)
    blk = scores_ref[...]
    rows = blk.shape[0]
    sub = scores_ref[:, pl.ds(i * rows, rows)]
    r_iota = jax.lax.broadcasted_iota(jnp.int32, (rows, rows), 0)
    c_iota = jax.lax.broadcasted_iota(jnp.int32, (rows, rows), 1)
    eq = r_iota == c_iota
    diag = jnp.max(jnp.where(eq, sub, -jnp.inf), axis=1, keepdims=True)
    thresh = FILTER_THRESHOLD * diag
    out_ref[...] = jnp.where(blk > thresh, blk * FILTER_FACTOR, blk)
    sub_filtered = jnp.where(sub > thresh, sub * FILTER_FACTOR, sub)
    out_ref[:, pl.ds(i * rows, rows)] = jnp.where(eq, sub, sub_filtered)


def kernel(scores):
    B = scores.shape[0]
    rows = _ROWS_PER_BLOCK
    grid = (B // rows,)
    return pl.pallas_call(
        _filter_block,
        grid=grid,
        in_specs=[pl.BlockSpec((rows, B), lambda i: (i, 0))],
        out_specs=pl.BlockSpec((rows, B), lambda i: (i, 0)),
        out_shape=jax.ShapeDtypeStruct(scores.shape, scores.dtype),
    )(scores)
